# baseline (speedup 1.0000x reference)
import functools

import jax
import jax.numpy as jnp
import numpy as np
from jax import lax
from jax.experimental import pallas as pl
from jax.experimental.pallas import tpu as pltpu
from jax.experimental.pallas import tpu_sc as plsc

VOCAB_SIZE = 1000000
DIM_MODEL = 64
POSITIONAL_ENCODING_ANGLE_BASE = 10000
POSITIONAL_ENCODING_LENGTH = 2048


def _positional_encoding_np(dim_model, angle_base=POSITIONAL_ENCODING_ANGLE_BASE,
                            length=POSITIONAL_ENCODING_LENGTH):
    depth = dim_model / 2
    positions = np.arange(length)[:, np.newaxis]
    depths = np.arange(depth)[np.newaxis, :]
    angle_rates = 1 / angle_base ** depths
    angle_rads = positions * angle_rates
    return np.concatenate([np.sin(angle_rads), np.cos(angle_rads)],
                          axis=-1).astype(np.float32)


_NW = 32
_LANES = 16
_NBUF = 2
_BBLK = 128
_UNROLL = 8


@functools.partial(jax.jit, static_argnames=("batch", "seq_len"))
def _sc_embed(idx, pe, table, *, batch, seq_len):
    dim = 64
    vregs_per_row = dim // _LANES

    mesh = plsc.VectorSubcoreMesh(core_axis_name="c", subcore_axis_name="s")

    @functools.partial(
        pl.kernel,
        out_type=jax.ShapeDtypeStruct((batch, seq_len, dim), jnp.float32),
        mesh=mesh,
        scratch_types=[
            [pltpu.VMEM((_BBLK,), jnp.int32) for _ in range(_NBUF)],
            [pltpu.VMEM((_BBLK,), jnp.int32) for _ in range(_NBUF)],
            [pltpu.VMEM((_BBLK, 2 * dim), jnp.float32) for _ in range(_NBUF)],
            [pltpu.VMEM((_BBLK, dim), jnp.float32) for _ in range(_NBUF)],
            pltpu.VMEM((seq_len * dim,), jnp.float32),
            [pltpu.SemaphoreType.DMA for _ in range(_NBUF)],
            [pltpu.SemaphoreType.DMA for _ in range(_NBUF)],
        ],
        compiler_params=pltpu.CompilerParams(use_tc_tiling_on_sc=True,
                                             needs_layout_passes=False),
    )
    def body(idx_hbm, pe_hbm, table_hbm, out_hbm,
             ibuf, pbuf, gbuf, stage, pe_v, gsem, ssem):
        wid = lax.axis_index("s") * 2 + lax.axis_index("c")
        b0 = wid * _BBLK
        one = jnp.full((_LANES,), 1, jnp.int32)

        pltpu.sync_copy(pe_hbm, pe_v)

        def load_idx(k, b):
            pltpu.sync_copy(
                idx_hbm.at[pl.ds(k * batch + b0, _BBLK)], ibuf[b])
            for g in range(_BBLK // _LANES):
                pbuf[b][pl.ds(g * _LANES, _LANES)] = lax.shift_right_logical(
                    ibuf[b][pl.ds(g * _LANES, _LANES)], one)

        for b in range(_NBUF):
            load_idx(b, b)
            pltpu.async_copy(table_hbm.at[pbuf[b]], gbuf[b], gsem[b])

        def pair(i, _):
            for b in range(_NBUF):
                s = i * _NBUF + b
                pltpu.make_async_copy(table_hbm.at[pbuf[b]], gbuf[b],
                                      gsem[b]).wait()

                @pl.when(i >= 1)
                def _():
                    pltpu.make_async_copy(
                        stage[b], out_hbm.at[pl.ds(b0, _BBLK), s - _NBUF],
                        ssem[b]).wait()

                pvals = [pe_v[pl.ds(s * dim + c * _LANES, _LANES)]
                         for c in range(vregs_per_row)]

                def fma_rows(r0, _):
                    hv = lax.bitwise_and(
                        ibuf[b][pl.ds(r0 * _LANES, _LANES)],
                        jnp.full((_LANES,), 1, jnp.int32)) * dim
                    for u in range(_LANES):
                        r = r0 * _LANES + u
                        h = hv[u]
                        for c in range(vregs_per_row):
                            v = gbuf[b][r, pl.ds(h + c * _LANES, _LANES)]
                            stage[b][r, pl.ds(c * _LANES, _LANES)] = (
                                v * jnp.float32(8.0) + pvals[c])
                    return 0

                lax.fori_loop(0, _BBLK // _LANES, fma_rows, 0)

                @pl.when(s + _NBUF < seq_len)
                def _():
                    load_idx(s + _NBUF, b)
                    pltpu.async_copy(table_hbm.at[pbuf[b]], gbuf[b], gsem[b])

                pltpu.async_copy(stage[b],
                                 out_hbm.at[pl.ds(b0, _BBLK), s],
                                 ssem[b])
            return 0

        lax.fori_loop(0, seq_len // _NBUF, pair, 0)

        for b in range(_NBUF):
            last = seq_len - _NBUF + b
            pltpu.make_async_copy(
                stage[b], out_hbm.at[pl.ds(b0, _BBLK), last],
                ssem[b]).wait()

    return body(idx, pe, table)


_PE_FULL = _positional_encoding_np(DIM_MODEL)


def kernel(x, table):
    batch, seq_len = x.shape
    idx = x.T.reshape(-1).astype(jnp.int32)
    pe = jnp.asarray(_PE_FULL[:seq_len]).reshape(-1)
    table2 = table.reshape(table.shape[0] // 2, 2 * table.shape[1])
    return _sc_embed(idx, pe, table2, batch=batch, seq_len=seq_len)


# speedup vs baseline: 1.2208x; 1.2208x over previous
import functools

import jax
import jax.numpy as jnp
import numpy as np
from jax import lax
from jax.experimental import pallas as pl
from jax.experimental.pallas import tpu as pltpu
from jax.experimental.pallas import tpu_sc as plsc

VOCAB_SIZE = 1000000
DIM_MODEL = 64
POSITIONAL_ENCODING_ANGLE_BASE = 10000
POSITIONAL_ENCODING_LENGTH = 2048


def _positional_encoding_np(dim_model, angle_base=POSITIONAL_ENCODING_ANGLE_BASE,
                            length=POSITIONAL_ENCODING_LENGTH):
    depth = dim_model / 2
    positions = np.arange(length)[:, np.newaxis]
    depths = np.arange(depth)[np.newaxis, :]
    angle_rates = 1 / angle_base ** depths
    angle_rads = positions * angle_rates
    return np.concatenate([np.sin(angle_rads), np.cos(angle_rads)],
                          axis=-1).astype(np.float32)


_NW = 32
_LANES = 16
_NBUF = 2
_BBLK = 128
_UNROLL = 8


@functools.partial(jax.jit, static_argnames=("batch", "seq_len"))
def _sc_embed(idx, pe, table, *, batch, seq_len):
    dim = 64
    vregs_per_row = dim // _LANES

    mesh = plsc.VectorSubcoreMesh(core_axis_name="c", subcore_axis_name="s")

    @functools.partial(
        pl.kernel,
        out_type=jax.ShapeDtypeStruct((batch, seq_len, dim), jnp.float32),
        mesh=mesh,
        scratch_types=[
            [pltpu.VMEM((_BBLK,), jnp.int32) for _ in range(_NBUF)],
            [pltpu.VMEM((_BBLK, dim), jnp.float32) for _ in range(_NBUF)],
            [pltpu.VMEM((_BBLK, dim), jnp.float32) for _ in range(_NBUF)],
            pltpu.VMEM((seq_len, dim), jnp.float32),
            [pltpu.SemaphoreType.DMA for _ in range(_NBUF)],
            [pltpu.SemaphoreType.DMA for _ in range(_NBUF)],
        ],
        compiler_params=pltpu.CompilerParams(use_tc_tiling_on_sc=False,
                                             needs_layout_passes=False),
    )
    def body(idx_hbm, pe_hbm, table_hbm, out_hbm,
             ibuf, gbuf, stage, pe_v, gsem, ssem):
        wid = lax.axis_index("s") * 2 + lax.axis_index("c")
        b0 = wid * _BBLK

        pltpu.sync_copy(pe_hbm, pe_v)

        def load_idx(k, b):
            pltpu.sync_copy(
                idx_hbm.at[pl.ds(k * batch + b0, _BBLK)], ibuf[b])

        for b in range(_NBUF):
            load_idx(b, b)
            pltpu.async_copy(table_hbm.at[ibuf[b]], gbuf[b], gsem[b])

        def pair(i, _):
            for b in range(_NBUF):
                s = i * _NBUF + b
                pltpu.make_async_copy(table_hbm.at[ibuf[b]], gbuf[b],
                                      gsem[b]).wait()

                @pl.when(i >= 1)
                def _():
                    pltpu.make_async_copy(
                        stage[b], out_hbm.at[pl.ds(b0, _BBLK), s - _NBUF],
                        ssem[b]).wait()

                pvals = [pe_v[s, pl.ds(c * _LANES, _LANES)]
                         for c in range(vregs_per_row)]

                def fma_rows(r0, _):
                    for u in range(_UNROLL):
                        r = r0 * _UNROLL + u
                        for c in range(vregs_per_row):
                            v = gbuf[b][r, pl.ds(c * _LANES, _LANES)]
                            stage[b][r, pl.ds(c * _LANES, _LANES)] = (
                                v * jnp.float32(8.0) + pvals[c])
                    return 0

                lax.fori_loop(0, _BBLK // _UNROLL, fma_rows, 0)

                @pl.when(s + _NBUF < seq_len)
                def _():
                    load_idx(s + _NBUF, b)
                    pltpu.async_copy(table_hbm.at[ibuf[b]], gbuf[b], gsem[b])

                pltpu.async_copy(stage[b],
                                 out_hbm.at[pl.ds(b0, _BBLK), s],
                                 ssem[b])
            return 0

        lax.fori_loop(0, seq_len // _NBUF, pair, 0)

        for b in range(_NBUF):
            last = seq_len - _NBUF + b
            pltpu.make_async_copy(
                stage[b], out_hbm.at[pl.ds(b0, _BBLK), last],
                ssem[b]).wait()

    return body(idx, pe, table)


_PE_FULL = _positional_encoding_np(DIM_MODEL)


def kernel(x, table):
    batch, seq_len = x.shape
    idx = x.T.reshape(-1).astype(jnp.int32)
    pe = jnp.asarray(_PE_FULL[:seq_len])
    return _sc_embed(idx, pe, table, batch=batch, seq_len=seq_len)
